# baseline (speedup 1.0000x reference)
import functools

import jax
import jax.numpy as jnp
from jax import lax
from jax.experimental import pallas as pl
from jax.experimental.pallas import tpu as pltpu
from jax.experimental.pallas import tpu_sc as plsc

VOCAB = 101
DIM = 128
BATCH = 16384

_info = plsc.get_sparse_core_info()
_NC = _info.num_cores
_NS = _info.num_subcores
_NW = _NC * _NS
_BPW = BATCH // _NW


_NB = 8
_CH = _BPW // _NB


@functools.partial(
    pl.kernel,
    mesh=plsc.VectorSubcoreMesh(core_axis_name="c", subcore_axis_name="s"),
    out_type=jax.ShapeDtypeStruct((BATCH, DIM), jnp.float32),
    scratch_types=[
        pltpu.VMEM((_BPW,), jnp.int32),
        pltpu.VMEM_SHARED((VOCAB, DIM), jnp.float32),
        pltpu.VMEM((_BPW, DIM), jnp.float32),
        pltpu.SemaphoreType.DMA,
        pltpu.SemaphoreType.DMA((_NB,)),
        pltpu.SemaphoreType.DMA((_NB,)),
    ],
)
def _embed_gather(table_hbm, idx_hbm, out_hbm, idx_v, table_sh, rows_v,
                  isem, gsem, ssem):
    sid = lax.axis_index("s")
    wid = sid * _NC + lax.axis_index("c")
    base = wid * _BPW
    icp = pltpu.async_copy(idx_hbm.at[pl.ds(base, _BPW)], idx_v, isem)
    for k, (s, n) in enumerate(((0, 32), (32, 32), (64, 32), (96, VOCAB - 96))):
        @pl.when(sid == k)
        def _(s=s, n=n):
            pltpu.sync_copy(table_hbm.at[pl.ds(s, n)], table_sh.at[pl.ds(s, n)])

    icp.wait()
    gcps = [
        pltpu.async_copy(
            table_hbm.at[idx_v.at[pl.ds(0, _CH)]],
            rows_v.at[pl.ds(0, _CH)],
            gsem.at[0],
        )
    ]
    plsc.subcore_barrier()
    gcps += [
        pltpu.async_copy(
            table_sh.at[idx_v.at[pl.ds(c * _CH, _CH)]],
            rows_v.at[pl.ds(c * _CH, _CH)],
            gsem.at[c],
        )
        for c in range(1, _NB)
    ]
    scps = []
    for c in range(_NB):
        gcps[c].wait()
        scps.append(
            pltpu.async_copy(
                rows_v.at[pl.ds(c * _CH, _CH)],
                out_hbm.at[pl.ds(base + c * _CH, _CH)],
                ssem.at[c],
            )
        )
    for scp in scps:
        scp.wait()


def kernel(batch_labels, embedding_table):
    idx = batch_labels.astype(jnp.int32)
    return _embed_gather(embedding_table, idx)


# speedup vs baseline: 1.1545x; 1.1545x over previous
import functools

import jax
import jax.numpy as jnp
from jax import lax
from jax.experimental import pallas as pl
from jax.experimental.pallas import tpu as pltpu
from jax.experimental.pallas import tpu_sc as plsc

VOCAB = 101
DIM = 128
BATCH = 16384

_info = plsc.get_sparse_core_info()
_NC = _info.num_cores
_NS = _info.num_subcores
_NW = _NC * _NS
_BPW = BATCH // _NW


_NB = 8
_CH = _BPW // _NB


@functools.partial(
    pl.kernel,
    mesh=plsc.VectorSubcoreMesh(core_axis_name="c", subcore_axis_name="s"),
    compiler_params=pltpu.CompilerParams(use_tc_tiling_on_sc=False),
    out_type=jax.ShapeDtypeStruct((BATCH, DIM), jnp.float32),
    scratch_types=[
        pltpu.VMEM((_BPW,), jnp.int32),
        pltpu.VMEM_SHARED((VOCAB, DIM), jnp.float32),
        pltpu.VMEM((_BPW, DIM), jnp.float32),
        pltpu.SemaphoreType.DMA,
        pltpu.SemaphoreType.DMA((_NB,)),
        pltpu.SemaphoreType.DMA((_NB,)),
    ],
)
def _embed_gather(table_hbm, idx_hbm, out_hbm, idx_v, table_sh, rows_v,
                  isem, gsem, ssem):
    sid = lax.axis_index("s")
    wid = sid * _NC + lax.axis_index("c")
    base = wid * _BPW
    icp = pltpu.async_copy(idx_hbm.at[pl.ds(base, _BPW)], idx_v, isem)
    for k, (s, n) in enumerate(((0, 32), (32, 32), (64, 32), (96, VOCAB - 96))):
        @pl.when(sid == k)
        def _(s=s, n=n):
            pltpu.sync_copy(table_hbm.at[pl.ds(s, n)], table_sh.at[pl.ds(s, n)])

    plsc.subcore_barrier()
    icp.wait()
    gcps = [
        pltpu.async_copy(
            table_sh.at[idx_v.at[pl.ds(c * _CH, _CH)]],
            rows_v.at[pl.ds(c * _CH, _CH)],
            gsem.at[c],
        )
        for c in range(_NB)
    ]
    scps = []
    for c in range(_NB):
        gcps[c].wait()
        scps.append(
            pltpu.async_copy(
                rows_v.at[pl.ds(c * _CH, _CH)],
                out_hbm.at[pl.ds(base + c * _CH, _CH)],
                ssem.at[c],
            )
        )
    for scp in scps:
        scp.wait()


def kernel(batch_labels, embedding_table):
    idx = batch_labels.astype(jnp.int32)
    return _embed_gather(embedding_table, idx)
